# Initial kernel scaffold; baseline (speedup 1.0000x reference)
import jax
import jax.numpy as jnp
from jax.experimental import pallas as pl


def kernel(feat, edge_index, W, b):
    raise NotImplementedError("write your pallas kernel here")


# speedup vs baseline: 3.2687x; 3.2687x over previous
import functools

import jax
import jax.numpy as jnp
from jax import lax
from jax.experimental import pallas as pl
from jax.experimental.pallas import tpu as pltpu
from jax.experimental.pallas import tpu_sc as plsc

N = 10000
E = 320000
D = 128

NC = 2
NS = 16
NW = NC * NS

CHUNK = 128
CH_PER_W = 80
EP = NW * CH_PER_W * CHUNK
NPAD = 10112
ROWS_PER_S = NPAD // NS


def _mm_body(feat_ref, w_ref, b_ref, o_ref):
  o_ref[...] = (
      jnp.dot(feat_ref[...], w_ref[...], preferred_element_type=jnp.float32)
      + b_ref[...]
  )


def _add_body(a_ref, b_ref, o_ref):
  o_ref[...] = a_ref[...] + b_ref[...]


_sc_mesh = plsc.VectorSubcoreMesh(core_axis_name="c", subcore_axis_name="s")


@functools.partial(
    pl.kernel,
    out_type=jax.ShapeDtypeStruct((NC, NPAD, D), jnp.float32),
    mesh=_sc_mesh,
    scratch_types=[
        pltpu.VMEM((CH_PER_W, CHUNK), jnp.int32),
        pltpu.VMEM((CH_PER_W, CHUNK), jnp.int32),
        pltpu.VMEM((CHUNK, D), jnp.float32),
        pltpu.VMEM_SHARED((NPAD, D), jnp.float32),
        pltpu.SemaphoreType.DMA,
    ],
)
def _sc_aggregate(src_hbm, dst_hbm, h_hbm, z_hbm, out_hbm,
                  src_v, dst_v, rows_v, acc, sem):
  c = lax.axis_index("c")
  s = lax.axis_index("s")
  wid = c * NS + s

  pltpu.sync_copy(z_hbm.at[pl.ds(s * ROWS_PER_S, ROWS_PER_S)],
                  acc.at[pl.ds(s * ROWS_PER_S, ROWS_PER_S)])
  pltpu.sync_copy(src_hbm.at[pl.ds(wid * CH_PER_W, CH_PER_W)], src_v)
  pltpu.sync_copy(dst_hbm.at[pl.ds(wid * CH_PER_W, CH_PER_W)], dst_v)
  plsc.subcore_barrier()

  def body(j, carry):
    pltpu.async_copy(h_hbm.at[src_v.at[j]], rows_v, sem).wait()
    pltpu.sync_copy(rows_v, acc.at[dst_v.at[j]], add=True)
    return carry

  lax.fori_loop(0, CH_PER_W, body, 0)
  plsc.subcore_barrier()
  pltpu.sync_copy(acc.at[pl.ds(s * ROWS_PER_S, ROWS_PER_S)],
                  out_hbm.at[c, pl.ds(s * ROWS_PER_S, ROWS_PER_S)])


def kernel(feat, edge_index, W, b):
  src = edge_index[0].astype(jnp.int32)
  dst = edge_index[1].astype(jnp.int32)
  pad = EP - E
  srcp = jnp.concatenate([src, jnp.zeros((pad,), jnp.int32)]).reshape(-1, CHUNK)
  dstp = jnp.concatenate([dst, jnp.full((pad,), N, jnp.int32)]).reshape(-1, CHUNK)

  h = pl.pallas_call(
      _mm_body,
      grid=(10,),
      in_specs=[
          pl.BlockSpec((N // 10, D), lambda i: (i, 0)),
          pl.BlockSpec((D, D), lambda i: (0, 0)),
          pl.BlockSpec((1, D), lambda i: (0, 0)),
      ],
      out_specs=pl.BlockSpec((N // 10, D), lambda i: (i, 0)),
      out_shape=jax.ShapeDtypeStruct((N, D), jnp.float32),
  )(feat, W, b.reshape(1, D))

  zeros = jnp.zeros((NPAD, D), jnp.float32)
  partials = _sc_aggregate(srcp, dstp, h, zeros)

  out = pl.pallas_call(
      _add_body,
      grid=(10,),
      in_specs=[
          pl.BlockSpec((N // 10, D), lambda i: (i, 0)),
          pl.BlockSpec((N // 10, D), lambda i: (i, 0)),
      ],
      out_specs=pl.BlockSpec((N // 10, D), lambda i: (i, 0)),
      out_shape=jax.ShapeDtypeStruct((N, D), jnp.float32),
  )(partials[0, :N], partials[1, :N])
  return out
